# baseline (speedup 1.0000x reference)
import functools

import jax
import jax.numpy as jnp
from jax import lax
from jax.experimental import pallas as pl
from jax.experimental.pallas import tpu as pltpu
from jax.experimental.pallas import tpu_sc as plsc

N_NODES = 10000
N_PAD = 10240
FEAT = 128
N_EDGES = 320000
NC = 2
NS = 16
NW = NC * NS
CHUNK = 80
NCHUNKS = 128
PHCH = 64
EPW = NCHUNKS * CHUNK
E_PAD = NW * EPW
RPT = N_PAD // NS
ZCH = 64


def _fill(ref, val, nrows, ncols):
    val16 = jnp.full((16,), val, jnp.float32)
    if ncols is None:
        def body(i, _):
            ref[pl.ds(i * 16, 16)] = val16
            return 0
        lax.fori_loop(0, nrows // 16, body, 0)
    else:
        def row(i, _):
            def col(j, _):
                ref[i, pl.ds(j * 16, 16)] = val16
                return 0
            return lax.fori_loop(0, ncols // 16, col, 0)
        lax.fori_loop(0, nrows, row, 0)


def _make_agg(with_cnt: bool):
    mesh = plsc.VectorSubcoreMesh(core_axis_name="c", subcore_axis_name="s")
    out_type = [jax.ShapeDtypeStruct((NC, N_PAD, FEAT), jnp.float32)]
    scratch = [
        pltpu.VMEM((PHCH, CHUNK), jnp.int32),
        pltpu.VMEM((PHCH, CHUNK), jnp.int32),
        pltpu.VMEM((CHUNK, FEAT), jnp.float32),
        pltpu.VMEM((CHUNK, FEAT), jnp.float32),
        pltpu.VMEM_SHARED((N_PAD, FEAT), jnp.float32),
        pltpu.SemaphoreType.DMA,
        pltpu.SemaphoreType.DMA,
    ]
    if with_cnt:
        out_type.append(jax.ShapeDtypeStruct((NC, N_PAD), jnp.float32))
        scratch += [
            pltpu.VMEM((CHUNK,), jnp.float32),
            pltpu.VMEM((RPT,), jnp.float32),
            pltpu.VMEM_SHARED((N_PAD,), jnp.float32),
        ]

    def body(feat, src, dst, *rest):
        if with_cnt:
            (out_sum, out_cnt, didx, sidx, rows_a, rows_b, ssum, sem_a,
             sem_b, ones, zc, scnt) = rest
        else:
            out_sum, didx, sidx, rows_a, rows_b, ssum, sem_a, sem_b = rest
        cid = lax.axis_index("c")
        sid = lax.axis_index("s")
        wid = cid * NS + sid
        rb = sid * RPT

        zbuf = rows_a.at[pl.ds(0, ZCH)]
        _fill(zbuf, 0.0, ZCH, FEAT)
        for k in range(RPT // ZCH):
            pltpu.sync_copy(zbuf, ssum.at[pl.ds(rb + k * ZCH, ZCH), :])
        if with_cnt:
            _fill(ones, 1.0, CHUNK, None)
            _fill(zc, 0.0, RPT, None)
            pltpu.sync_copy(zc, scnt.at[pl.ds(rb, RPT)])

        plsc.subcore_barrier()

        def gather(i, rows, sem):
            pltpu.async_copy(feat.at[sidx.at[i]], rows, sem)

        def gather_wait(i, rows, sem):
            pltpu.make_async_copy(feat.at[sidx.at[i]], rows, sem).wait()

        def scatter(i, rows):
            pltpu.sync_copy(rows, ssum.at[didx.at[i]], add=True)
            if with_cnt:
                pltpu.sync_copy(ones, scnt.at[didx.at[i]], add=True)

        for base in (0, PHCH):
            pltpu.sync_copy(dst.at[wid, pl.ds(base, PHCH)], didx)
            pltpu.sync_copy(src.at[wid, pl.ds(base, PHCH)], sidx)
            gather(0, rows_a, sem_a)

            def pair(j, _):
                i0 = 2 * j
                gather_wait(i0, rows_a, sem_a)
                gather(i0 + 1, rows_b, sem_b)
                scatter(i0, rows_a)
                gather_wait(i0 + 1, rows_b, sem_b)

                @pl.when(i0 + 2 < PHCH)
                def _():
                    gather(i0 + 2, rows_a, sem_a)

                scatter(i0 + 1, rows_b)
                return 0

            lax.fori_loop(0, PHCH // 2, pair, 0)
        plsc.subcore_barrier()

        pltpu.sync_copy(ssum.at[pl.ds(rb, RPT), :],
                        out_sum.at[cid, pl.ds(rb, RPT), :])
        if with_cnt:
            pltpu.sync_copy(scnt.at[pl.ds(rb, RPT)],
                            out_cnt.at[cid, pl.ds(rb, RPT)])

    return pl.kernel(body, out_type=out_type, mesh=mesh,
                     scratch_types=scratch)


_agg_cnt = _make_agg(True)
_agg = _make_agg(False)

BR = 2048


def _dense_body(relu):
    def body(sp_ref, cp_ref, x_ref, wl_ref, wr_ref, b_ref, o_ref):
        c = jnp.clip(cp_ref[0] + cp_ref[1], 1.0, None)
        mean = (sp_ref[0] + sp_ref[1]) / c[:, None]
        acc = jnp.dot(mean, wl_ref[...], preferred_element_type=jnp.float32)
        acc = acc + jnp.dot(x_ref[...], wr_ref[...],
                            preferred_element_type=jnp.float32)
        acc = acc + b_ref[...]
        if relu:
            acc = jnp.maximum(acc, 0.0)
        o_ref[...] = acc
    return body


def _dense_layer(sp, cp, x, wl, wr, b, relu):
    return pl.pallas_call(
        _dense_body(relu),
        grid=(N_PAD // BR,),
        in_specs=[
            pl.BlockSpec((NC, BR, FEAT), lambda i: (0, i, 0)),
            pl.BlockSpec((NC, BR), lambda i: (0, i)),
            pl.BlockSpec((BR, FEAT), lambda i: (i, 0)),
            pl.BlockSpec((FEAT, FEAT), lambda i: (0, 0)),
            pl.BlockSpec((FEAT, FEAT), lambda i: (0, 0)),
            pl.BlockSpec((1, FEAT), lambda i: (0, 0)),
        ],
        out_specs=pl.BlockSpec((BR, FEAT), lambda i: (i, 0)),
        out_shape=jax.ShapeDtypeStruct((N_PAD, FEAT), jnp.float32),
    )(sp, cp, x, wl, wr, b)


def kernel(x, edge_index, Wl1, Wr1, b1, Wl2, Wr2, b2):
    ei = edge_index.astype(jnp.int32)
    pad = jnp.full((2, E_PAD - N_EDGES), N_NODES, jnp.int32)
    ei = jnp.concatenate([ei, pad], axis=1)
    src = ei[0].reshape(NW, NCHUNKS, CHUNK)
    dst = ei[1].reshape(NW, NCHUNKS, CHUNK)
    x_pad = jnp.pad(x, ((0, N_PAD - N_NODES), (0, 0)))
    sp1, cp = _agg_cnt(x_pad, src, dst)
    h = _dense_layer(sp1, cp, x_pad, Wl1, Wr1, b1.reshape(1, FEAT), True)
    sp2, = _agg(h, src, dst)
    out = _dense_layer(sp2, cp, h, Wl2, Wr2, b2.reshape(1, FEAT), False)
    return out[:N_NODES]


# speedup vs baseline: 1.0454x; 1.0454x over previous
import functools

import jax
import jax.numpy as jnp
from jax import lax
from jax.experimental import pallas as pl
from jax.experimental.pallas import tpu as pltpu
from jax.experimental.pallas import tpu_sc as plsc

N_NODES = 10000
N_PAD = 10240
FEAT = 128
N_EDGES = 320000
NC = 2
NS = 16
NW = NC * NS
CHUNK = 80
NCHUNKS = 128
PHCH = 64
UNROLL = 8
EPW = NCHUNKS * CHUNK
E_PAD = NW * EPW
RPT = N_PAD // NS
ZCH = 64


def _fill(ref, val, nrows, ncols):
    val16 = jnp.full((16,), val, jnp.float32)
    if ncols is None:
        def body(i, _):
            ref[pl.ds(i * 16, 16)] = val16
            return 0
        lax.fori_loop(0, nrows // 16, body, 0)
    else:
        def row(i, _):
            def col(j, _):
                ref[i, pl.ds(j * 16, 16)] = val16
                return 0
            return lax.fori_loop(0, ncols // 16, col, 0)
        lax.fori_loop(0, nrows, row, 0)


def _make_agg(with_cnt: bool):
    mesh = plsc.VectorSubcoreMesh(core_axis_name="c", subcore_axis_name="s")
    out_type = [jax.ShapeDtypeStruct((NC, N_PAD, FEAT), jnp.float32)]
    scratch = [
        pltpu.VMEM((PHCH, CHUNK), jnp.int32),
        pltpu.VMEM((PHCH, CHUNK), jnp.int32),
        pltpu.VMEM((CHUNK, FEAT), jnp.float32),
        pltpu.VMEM((CHUNK, FEAT), jnp.float32),
        pltpu.VMEM_SHARED((N_PAD, FEAT), jnp.float32),
        pltpu.SemaphoreType.DMA,
        pltpu.SemaphoreType.DMA,
    ]
    if with_cnt:
        out_type.append(jax.ShapeDtypeStruct((NC, N_PAD), jnp.float32))
        scratch += [
            pltpu.VMEM((CHUNK,), jnp.float32),
            pltpu.VMEM((RPT,), jnp.float32),
            pltpu.VMEM_SHARED((N_PAD,), jnp.float32),
        ]

    def body(feat, src, dst, *rest):
        if with_cnt:
            (out_sum, out_cnt, didx, sidx, rows_a, rows_b, ssum, sem_a,
             sem_b, ones, zc, scnt) = rest
        else:
            out_sum, didx, sidx, rows_a, rows_b, ssum, sem_a, sem_b = rest
        cid = lax.axis_index("c")
        sid = lax.axis_index("s")
        wid = cid * NS + sid
        rb = sid * RPT

        zbuf = rows_a.at[pl.ds(0, ZCH)]
        _fill(zbuf, 0.0, ZCH, FEAT)
        for k in range(RPT // ZCH):
            pltpu.sync_copy(zbuf, ssum.at[pl.ds(rb + k * ZCH, ZCH), :])
        if with_cnt:
            _fill(ones, 1.0, CHUNK, None)
            _fill(zc, 0.0, RPT, None)
            pltpu.sync_copy(zc, scnt.at[pl.ds(rb, RPT)])

        plsc.subcore_barrier()

        def gather(i, rows, sem):
            return pltpu.async_copy(feat.at[sidx.at[i]], rows, sem)

        def scatter(i, rows):
            pltpu.sync_copy(rows, ssum.at[didx.at[i]], add=True)
            if with_cnt:
                pltpu.sync_copy(ones, scnt.at[didx.at[i]], add=True)

        for base in (0, PHCH):
            pltpu.sync_copy(dst.at[wid, pl.ds(base, PHCH)], didx)
            pltpu.sync_copy(src.at[wid, pl.ds(base, PHCH)], sidx)
            def group(g, _):
                go = g * UNROLL
                bufs = ((rows_a, sem_a), (rows_b, sem_b))
                descs = [None] * UNROLL
                descs[0] = gather(go, *bufs[0])
                for k in range(UNROLL):
                    if k + 1 < UNROLL:
                        descs[k + 1] = gather(go + k + 1, *bufs[(k + 1) % 2])
                    descs[k].wait()
                    scatter(go + k, bufs[k % 2][0])
                return 0

            lax.fori_loop(0, PHCH // UNROLL, group, 0)
        plsc.subcore_barrier()

        pltpu.sync_copy(ssum.at[pl.ds(rb, RPT), :],
                        out_sum.at[cid, pl.ds(rb, RPT), :])
        if with_cnt:
            pltpu.sync_copy(scnt.at[pl.ds(rb, RPT)],
                            out_cnt.at[cid, pl.ds(rb, RPT)])

    return pl.kernel(body, out_type=out_type, mesh=mesh,
                     scratch_types=scratch)


_agg_cnt = _make_agg(True)
_agg = _make_agg(False)

BR = 2048


def _dense_body(relu):
    def body(sp_ref, cp_ref, x_ref, wl_ref, wr_ref, b_ref, o_ref):
        c = jnp.clip(cp_ref[0] + cp_ref[1], 1.0, None)
        mean = (sp_ref[0] + sp_ref[1]) / c[:, None]
        acc = jnp.dot(mean, wl_ref[...], preferred_element_type=jnp.float32)
        acc = acc + jnp.dot(x_ref[...], wr_ref[...],
                            preferred_element_type=jnp.float32)
        acc = acc + b_ref[...]
        if relu:
            acc = jnp.maximum(acc, 0.0)
        o_ref[...] = acc
    return body


def _dense_layer(sp, cp, x, wl, wr, b, relu):
    return pl.pallas_call(
        _dense_body(relu),
        grid=(N_PAD // BR,),
        in_specs=[
            pl.BlockSpec((NC, BR, FEAT), lambda i: (0, i, 0)),
            pl.BlockSpec((NC, BR), lambda i: (0, i)),
            pl.BlockSpec((BR, FEAT), lambda i: (i, 0)),
            pl.BlockSpec((FEAT, FEAT), lambda i: (0, 0)),
            pl.BlockSpec((FEAT, FEAT), lambda i: (0, 0)),
            pl.BlockSpec((1, FEAT), lambda i: (0, 0)),
        ],
        out_specs=pl.BlockSpec((BR, FEAT), lambda i: (i, 0)),
        out_shape=jax.ShapeDtypeStruct((N_PAD, FEAT), jnp.float32),
    )(sp, cp, x, wl, wr, b)


def kernel(x, edge_index, Wl1, Wr1, b1, Wl2, Wr2, b2):
    ei = edge_index.astype(jnp.int32)
    pad = jnp.full((2, E_PAD - N_EDGES), N_NODES, jnp.int32)
    ei = jnp.concatenate([ei, pad], axis=1)
    src = ei[0].reshape(NW, NCHUNKS, CHUNK)
    dst = ei[1].reshape(NW, NCHUNKS, CHUNK)
    x_pad = jnp.pad(x, ((0, N_PAD - N_NODES), (0, 0)))
    sp1, cp = _agg_cnt(x_pad, src, dst)
    h = _dense_layer(sp1, cp, x_pad, Wl1, Wr1, b1.reshape(1, FEAT), True)
    sp2, = _agg(h, src, dst)
    out = _dense_layer(sp2, cp, h, Wl2, Wr2, b2.reshape(1, FEAT), False)
    return out[:N_NODES]


# speedup vs baseline: 1.0510x; 1.0054x over previous
import functools

import jax
import jax.numpy as jnp
from jax import lax
from jax.experimental import pallas as pl
from jax.experimental.pallas import tpu as pltpu
from jax.experimental.pallas import tpu_sc as plsc

N_NODES = 10000
N_PAD = 10240
FEAT = 128
N_EDGES = 320000
NC = 2
NS = 16
NW = NC * NS
CHUNK = 128
NCHUNKS = 80
PHCH = 40
EPW = NCHUNKS * CHUNK
E_PAD = NW * EPW
RPT = N_PAD // NS
ZCH = 128


def _fill(ref, val, nrows, ncols):
    val16 = jnp.full((16,), val, jnp.float32)
    if ncols is None:
        def body(i, _):
            ref[pl.ds(i * 16, 16)] = val16
            return 0
        lax.fori_loop(0, nrows // 16, body, 0)
    else:
        def row(i, _):
            def col(j, _):
                ref[i, pl.ds(j * 16, 16)] = val16
                return 0
            return lax.fori_loop(0, ncols // 16, col, 0)
        lax.fori_loop(0, nrows, row, 0)


def _make_agg(with_cnt: bool):
    mesh = plsc.VectorSubcoreMesh(core_axis_name="c", subcore_axis_name="s")
    out_type = [jax.ShapeDtypeStruct((NC, N_PAD, FEAT), jnp.float32)]
    scratch = [
        pltpu.VMEM((PHCH, CHUNK), jnp.int32),
        pltpu.VMEM((PHCH, CHUNK), jnp.int32),
        pltpu.VMEM((CHUNK, FEAT), jnp.float32),
        pltpu.VMEM_SHARED((N_PAD, FEAT), jnp.float32),
        pltpu.SemaphoreType.DMA,
    ]
    if with_cnt:
        out_type.append(jax.ShapeDtypeStruct((NC, N_PAD), jnp.float32))
        scratch += [
            pltpu.VMEM((CHUNK,), jnp.float32),
            pltpu.VMEM((RPT,), jnp.float32),
            pltpu.VMEM_SHARED((N_PAD,), jnp.float32),
        ]

    def body(feat, src, dst, *rest):
        if with_cnt:
            (out_sum, out_cnt, didx, sidx, rows, ssum, gsem,
             ones, zc, scnt) = rest
        else:
            out_sum, didx, sidx, rows, ssum, gsem = rest
        cid = lax.axis_index("c")
        sid = lax.axis_index("s")
        wid = cid * NS + sid
        rb = sid * RPT

        zbuf = rows.at[pl.ds(0, ZCH)]
        _fill(zbuf, 0.0, ZCH, FEAT)
        for k in range(RPT // ZCH):
            pltpu.sync_copy(zbuf, ssum.at[pl.ds(rb + k * ZCH, ZCH), :])
        if with_cnt:
            _fill(ones, 1.0, CHUNK, None)
            _fill(zc, 0.0, RPT, None)
            pltpu.sync_copy(zc, scnt.at[pl.ds(rb, RPT)])

        plsc.subcore_barrier()

        def gather(i, rows, sem):
            return pltpu.async_copy(feat.at[sidx.at[i]], rows, sem)

        def scatter(i, rows):
            pltpu.sync_copy(rows, ssum.at[didx.at[i]], add=True)
            if with_cnt:
                pltpu.sync_copy(ones, scnt.at[didx.at[i]], add=True)

        for base in (0, PHCH):
            pltpu.sync_copy(dst.at[wid, pl.ds(base, PHCH)], didx)
            pltpu.sync_copy(src.at[wid, pl.ds(base, PHCH)], sidx)

            def chunk(i, _):
                gather(i, rows, gsem).wait()
                scatter(i, rows)
                return 0

            lax.fori_loop(0, PHCH, chunk, 0)
        plsc.subcore_barrier()

        pltpu.sync_copy(ssum.at[pl.ds(rb, RPT), :],
                        out_sum.at[cid, pl.ds(rb, RPT), :])
        if with_cnt:
            pltpu.sync_copy(scnt.at[pl.ds(rb, RPT)],
                            out_cnt.at[cid, pl.ds(rb, RPT)])

    return pl.kernel(body, out_type=out_type, mesh=mesh,
                     scratch_types=scratch)


_agg_cnt = _make_agg(True)
_agg = _make_agg(False)

BR = 2048


def _dense_body(relu):
    def body(sp_ref, cp_ref, x_ref, wl_ref, wr_ref, b_ref, o_ref):
        c = jnp.clip(cp_ref[0] + cp_ref[1], 1.0, None)
        mean = (sp_ref[0] + sp_ref[1]) / c[:, None]
        acc = jnp.dot(mean, wl_ref[...], preferred_element_type=jnp.float32)
        acc = acc + jnp.dot(x_ref[...], wr_ref[...],
                            preferred_element_type=jnp.float32)
        acc = acc + b_ref[...]
        if relu:
            acc = jnp.maximum(acc, 0.0)
        o_ref[...] = acc
    return body


def _dense_layer(sp, cp, x, wl, wr, b, relu):
    return pl.pallas_call(
        _dense_body(relu),
        grid=(N_PAD // BR,),
        in_specs=[
            pl.BlockSpec((NC, BR, FEAT), lambda i: (0, i, 0)),
            pl.BlockSpec((NC, BR), lambda i: (0, i)),
            pl.BlockSpec((BR, FEAT), lambda i: (i, 0)),
            pl.BlockSpec((FEAT, FEAT), lambda i: (0, 0)),
            pl.BlockSpec((FEAT, FEAT), lambda i: (0, 0)),
            pl.BlockSpec((1, FEAT), lambda i: (0, 0)),
        ],
        out_specs=pl.BlockSpec((BR, FEAT), lambda i: (i, 0)),
        out_shape=jax.ShapeDtypeStruct((N_PAD, FEAT), jnp.float32),
    )(sp, cp, x, wl, wr, b)


def kernel(x, edge_index, Wl1, Wr1, b1, Wl2, Wr2, b2):
    ei = edge_index.astype(jnp.int32)
    pad = jnp.full((2, E_PAD - N_EDGES), N_NODES, jnp.int32)
    ei = jnp.concatenate([ei, pad], axis=1)
    src = ei[0].reshape(NW, NCHUNKS, CHUNK)
    dst = ei[1].reshape(NW, NCHUNKS, CHUNK)
    x_pad = jnp.pad(x, ((0, N_PAD - N_NODES), (0, 0)))
    sp1, cp = _agg_cnt(x_pad, src, dst)
    h = _dense_layer(sp1, cp, x_pad, Wl1, Wr1, b1.reshape(1, FEAT), True)
    sp2, = _agg(h, src, dst)
    out = _dense_layer(sp2, cp, h, Wl2, Wr2, b2.reshape(1, FEAT), False)
    return out[:N_NODES]


# speedup vs baseline: 2.5475x; 2.4239x over previous
import functools

import jax
import jax.numpy as jnp
from jax import lax
from jax.experimental import pallas as pl
from jax.experimental.pallas import tpu as pltpu
from jax.experimental.pallas import tpu_sc as plsc

N_NODES = 10000
N_PAD = 10240
FEAT = 128
N_EDGES = 320000
NC = 2
NS = 16
NW = NC * NS
CHUNK = 128
NCHUNKS = 80
PHCH = 40
EPW = NCHUNKS * CHUNK
E_PAD = NW * EPW
RPT = N_PAD // NS
ZCH = 128


def _fill(ref, val, nrows, ncols):
    val16 = jnp.full((16,), val, jnp.float32)
    if ncols is None:
        def body(i, _):
            ref[pl.ds(i * 16, 16)] = val16
            return 0
        lax.fori_loop(0, nrows // 16, body, 0)
    else:
        def row(i, _):
            def col(j, _):
                ref[i, pl.ds(j * 16, 16)] = val16
                return 0
            return lax.fori_loop(0, ncols // 16, col, 0)
        lax.fori_loop(0, nrows, row, 0)


def _make_agg(with_cnt: bool):
    mesh = plsc.VectorSubcoreMesh(core_axis_name="c", subcore_axis_name="s")
    out_type = [jax.ShapeDtypeStruct((NC, N_PAD, FEAT), jnp.float32)]
    scratch = [
        pltpu.VMEM((PHCH, CHUNK), jnp.int32),
        pltpu.VMEM((PHCH, CHUNK), jnp.int32),
        pltpu.VMEM((CHUNK, FEAT), jnp.float32),
        pltpu.VMEM_SHARED((N_PAD, FEAT), jnp.float32),
        pltpu.SemaphoreType.DMA,
    ]
    if with_cnt:
        out_type.append(jax.ShapeDtypeStruct((NC, N_PAD), jnp.float32))
        scratch += [
            pltpu.VMEM((CHUNK,), jnp.float32),
            pltpu.VMEM((RPT,), jnp.float32),
            pltpu.VMEM_SHARED((N_PAD,), jnp.float32),
        ]

    def body(feat, src, dst, *rest):
        if with_cnt:
            (out_sum, out_cnt, didx, sidx, rows, ssum, gsem,
             ones, zc, scnt) = rest
        else:
            out_sum, didx, sidx, rows, ssum, gsem = rest
        cid = lax.axis_index("c")
        sid = lax.axis_index("s")
        wid = cid * NS + sid
        rb = sid * RPT

        zbuf = rows.at[pl.ds(0, ZCH)]
        _fill(zbuf, 0.0, ZCH, FEAT)
        for k in range(RPT // ZCH):
            pltpu.sync_copy(zbuf, ssum.at[pl.ds(rb + k * ZCH, ZCH), :])
        if with_cnt:
            _fill(ones, 1.0, CHUNK, None)
            _fill(zc, 0.0, RPT, None)
            pltpu.sync_copy(zc, scnt.at[pl.ds(rb, RPT)])

        plsc.subcore_barrier()

        def gather(i, rows, sem):
            return pltpu.async_copy(feat.at[sidx.at[i]], rows, sem)

        def scatter(i, rows):
            pltpu.sync_copy(rows, ssum.at[didx.at[i]], add=True)
            if with_cnt:
                pltpu.sync_copy(ones, scnt.at[didx.at[i]], add=True)

        for base in (0, PHCH):
            pltpu.sync_copy(dst.at[wid, pl.ds(base, PHCH)], didx)
            pltpu.sync_copy(src.at[wid, pl.ds(base, PHCH)], sidx)

            def chunk(i, _):
                gather(i, rows, gsem).wait()
                scatter(i, rows)
                return 0

            lax.fori_loop(0, PHCH, chunk, 0)
        plsc.subcore_barrier()

        pltpu.sync_copy(ssum.at[pl.ds(rb, RPT), :],
                        out_sum.at[cid, pl.ds(rb, RPT), :])
        if with_cnt:
            pltpu.sync_copy(scnt.at[pl.ds(rb, RPT)],
                            out_cnt.at[cid, pl.ds(rb, RPT)])

    return pl.kernel(body, out_type=out_type, mesh=mesh,
                     scratch_types=scratch)


_agg_cnt = _make_agg(True)
_agg = _make_agg(False)

BR = 2048


def _dense_body(relu):
    def body(sp_ref, cp_ref, x_ref, wl_ref, wr_ref, b_ref, o_ref):
        c = jnp.clip(cp_ref[0] + cp_ref[1], 1.0, None)
        mean = (sp_ref[0] + sp_ref[1]) / c[:, None]
        acc = jnp.dot(mean, wl_ref[...], preferred_element_type=jnp.float32)
        acc = acc + jnp.dot(x_ref[...], wr_ref[...],
                            preferred_element_type=jnp.float32)
        acc = acc + b_ref[...]
        if relu:
            acc = jnp.maximum(acc, 0.0)
        o_ref[...] = acc
    return body


def _dense_layer(sp, cp, x, wl, wr, b, relu):
    return pl.pallas_call(
        _dense_body(relu),
        grid=(N_PAD // BR,),
        in_specs=[
            pl.BlockSpec((NC, BR, FEAT), lambda i: (0, i, 0)),
            pl.BlockSpec((NC, BR), lambda i: (0, i)),
            pl.BlockSpec((BR, FEAT), lambda i: (i, 0)),
            pl.BlockSpec((FEAT, FEAT), lambda i: (0, 0)),
            pl.BlockSpec((FEAT, FEAT), lambda i: (0, 0)),
            pl.BlockSpec((1, FEAT), lambda i: (0, 0)),
        ],
        out_specs=pl.BlockSpec((BR, FEAT), lambda i: (i, 0)),
        out_shape=jax.ShapeDtypeStruct((N_PAD, FEAT), jnp.float32),
    )(sp, cp, x, wl, wr, b)


def kernel(x, edge_index, Wl1, Wr1, b1, Wl2, Wr2, b2):
    ei = edge_index.astype(jnp.int32)
    rpw = N_EDGES // NW
    npad = EPW - rpw
    real = ei.reshape(2, NW, rpw)
    padv = jnp.broadcast_to(
        N_NODES + jnp.arange(npad, dtype=jnp.int32), (2, NW, npad))
    full = jnp.concatenate([real, padv], axis=2)
    src = full[0].reshape(NW, NCHUNKS, CHUNK)
    dst = full[1].reshape(NW, NCHUNKS, CHUNK)
    x_pad = jnp.pad(x, ((0, N_PAD - N_NODES), (0, 0)))
    sp1, cp = _agg_cnt(x_pad, src, dst)
    h = _dense_layer(sp1, cp, x_pad, Wl1, Wr1, b1.reshape(1, FEAT), True)
    sp2, = _agg(h, src, dst)
    out = _dense_layer(sp2, cp, h, Wl2, Wr2, b2.reshape(1, FEAT), False)
    return out[:N_NODES]


# speedup vs baseline: 3.2139x; 1.2616x over previous
import functools

import jax
import jax.numpy as jnp
from jax import lax
from jax.experimental import pallas as pl
from jax.experimental.pallas import tpu as pltpu
from jax.experimental.pallas import tpu_sc as plsc

N_NODES = 10000
N_PAD = 10240
FEAT = 128
N_EDGES = 320000
NC = 2
NS = 16
NW = NC * NS
CHUNK = 80
NCHUNKS = 128
PHCH = 64
UNROLL = 8
EPW = NCHUNKS * CHUNK
E_PAD = NW * EPW
RPT = N_PAD // NS
ZCH = 80


def _fill(ref, val, nrows, ncols):
    val16 = jnp.full((16,), val, jnp.float32)
    if ncols is None:
        def body(i, _):
            ref[pl.ds(i * 16, 16)] = val16
            return 0
        lax.fori_loop(0, nrows // 16, body, 0)
    else:
        def row(i, _):
            def col(j, _):
                ref[i, pl.ds(j * 16, 16)] = val16
                return 0
            return lax.fori_loop(0, ncols // 16, col, 0)
        lax.fori_loop(0, nrows, row, 0)


def _make_agg(with_cnt: bool):
    mesh = plsc.VectorSubcoreMesh(core_axis_name="c", subcore_axis_name="s")
    out_type = [jax.ShapeDtypeStruct((NC, N_PAD, FEAT), jnp.float32)]
    scratch = [
        pltpu.VMEM((PHCH, CHUNK), jnp.int32),
        pltpu.VMEM((PHCH, CHUNK), jnp.int32),
        pltpu.VMEM((CHUNK, FEAT), jnp.float32),
        pltpu.VMEM((CHUNK, FEAT), jnp.float32),
        pltpu.VMEM_SHARED((N_PAD, FEAT), jnp.float32),
        pltpu.SemaphoreType.DMA,
        pltpu.SemaphoreType.DMA,
    ]
    if with_cnt:
        out_type.append(jax.ShapeDtypeStruct((NC, N_PAD), jnp.float32))
        scratch += [
            pltpu.VMEM((CHUNK,), jnp.float32),
            pltpu.VMEM((RPT,), jnp.float32),
            pltpu.VMEM_SHARED((N_PAD,), jnp.float32),
        ]

    def body(feat, src, dst, *rest):
        if with_cnt:
            (out_sum, out_cnt, didx, sidx, rows_a, rows_b, ssum, sem_a,
             sem_b, ones, zc, scnt) = rest
        else:
            out_sum, didx, sidx, rows_a, rows_b, ssum, sem_a, sem_b = rest
        cid = lax.axis_index("c")
        sid = lax.axis_index("s")
        wid = cid * NS + sid
        rb = sid * RPT

        zbuf = rows_a.at[pl.ds(0, ZCH)]
        _fill(zbuf, 0.0, ZCH, FEAT)
        for k in range(RPT // ZCH):
            pltpu.sync_copy(zbuf, ssum.at[pl.ds(rb + k * ZCH, ZCH), :])
        if with_cnt:
            _fill(ones, 1.0, CHUNK, None)
            _fill(zc, 0.0, RPT, None)
            pltpu.sync_copy(zc, scnt.at[pl.ds(rb, RPT)])

        plsc.subcore_barrier()

        def gather(i, rows, sem):
            return pltpu.async_copy(feat.at[sidx.at[i]], rows, sem)

        def scatter(i, rows):
            pltpu.sync_copy(rows, ssum.at[didx.at[i]], add=True)
            if with_cnt:
                pltpu.sync_copy(ones, scnt.at[didx.at[i]], add=True)

        for base in (0, PHCH):
            pltpu.sync_copy(dst.at[wid, pl.ds(base, PHCH)], didx)
            pltpu.sync_copy(src.at[wid, pl.ds(base, PHCH)], sidx)

            def group(g, _):
                go = g * UNROLL
                bufs = ((rows_a, sem_a), (rows_b, sem_b))
                descs = [None] * UNROLL
                descs[0] = gather(go, *bufs[0])
                for k in range(UNROLL):
                    if k + 1 < UNROLL:
                        descs[k + 1] = gather(go + k + 1, *bufs[(k + 1) % 2])
                    descs[k].wait()
                    scatter(go + k, bufs[k % 2][0])
                return 0

            lax.fori_loop(0, PHCH // UNROLL, group, 0)
        plsc.subcore_barrier()

        pltpu.sync_copy(ssum.at[pl.ds(rb, RPT), :],
                        out_sum.at[cid, pl.ds(rb, RPT), :])
        if with_cnt:
            pltpu.sync_copy(scnt.at[pl.ds(rb, RPT)],
                            out_cnt.at[cid, pl.ds(rb, RPT)])

    return pl.kernel(body, out_type=out_type, mesh=mesh,
                     scratch_types=scratch)


_agg_cnt = _make_agg(True)
_agg = _make_agg(False)

BR = 2048


def _dense_body(relu):
    def body(sp_ref, cp_ref, x_ref, wl_ref, wr_ref, b_ref, o_ref):
        c = jnp.clip(cp_ref[0] + cp_ref[1], 1.0, None)
        mean = (sp_ref[0] + sp_ref[1]) / c[:, None]
        acc = jnp.dot(mean, wl_ref[...], preferred_element_type=jnp.float32)
        acc = acc + jnp.dot(x_ref[...], wr_ref[...],
                            preferred_element_type=jnp.float32)
        acc = acc + b_ref[...]
        if relu:
            acc = jnp.maximum(acc, 0.0)
        o_ref[...] = acc
    return body


def _dense_layer(sp, cp, x, wl, wr, b, relu):
    return pl.pallas_call(
        _dense_body(relu),
        grid=(N_PAD // BR,),
        in_specs=[
            pl.BlockSpec((NC, BR, FEAT), lambda i: (0, i, 0)),
            pl.BlockSpec((NC, BR), lambda i: (0, i)),
            pl.BlockSpec((BR, FEAT), lambda i: (i, 0)),
            pl.BlockSpec((FEAT, FEAT), lambda i: (0, 0)),
            pl.BlockSpec((FEAT, FEAT), lambda i: (0, 0)),
            pl.BlockSpec((1, FEAT), lambda i: (0, 0)),
        ],
        out_specs=pl.BlockSpec((BR, FEAT), lambda i: (i, 0)),
        out_shape=jax.ShapeDtypeStruct((N_PAD, FEAT), jnp.float32),
    )(sp, cp, x, wl, wr, b)


def kernel(x, edge_index, Wl1, Wr1, b1, Wl2, Wr2, b2):
    ei = edge_index.astype(jnp.int32)
    rpw = N_EDGES // NW
    npad = EPW - rpw
    real = ei.reshape(2, NW, rpw)
    padv = jnp.broadcast_to(
        N_NODES + jnp.arange(npad, dtype=jnp.int32), (2, NW, npad))
    full = jnp.concatenate([real, padv], axis=2)
    src = full[0].reshape(NW, NCHUNKS, CHUNK)
    dst = full[1].reshape(NW, NCHUNKS, CHUNK)
    x_pad = jnp.pad(x, ((0, N_PAD - N_NODES), (0, 0)))
    sp1, cp = _agg_cnt(x_pad, src, dst)
    h = _dense_layer(sp1, cp, x_pad, Wl1, Wr1, b1.reshape(1, FEAT), True)
    sp2, = _agg(h, src, dst)
    out = _dense_layer(sp2, cp, h, Wl2, Wr2, b2.reshape(1, FEAT), False)
    return out[:N_NODES]


# speedup vs baseline: 3.3385x; 1.0388x over previous
import functools

import jax
import jax.numpy as jnp
from jax import lax
from jax.experimental import pallas as pl
from jax.experimental.pallas import tpu as pltpu
from jax.experimental.pallas import tpu_sc as plsc

N_NODES = 10000
N_PAD = 10240
FEAT = 128
N_EDGES = 320000
NC = 2
NS = 16
NW = NC * NS
CHUNK = 80
NCHUNKS = 128
PHCH = 64
UNROLL = 16
EPW = NCHUNKS * CHUNK
E_PAD = NW * EPW
RPT = N_PAD // NS
ZCH = 80


def _fill(ref, val, nrows, ncols):
    val16 = jnp.full((16,), val, jnp.float32)
    if ncols is None:
        def body(i, _):
            ref[pl.ds(i * 16, 16)] = val16
            return 0
        lax.fori_loop(0, nrows // 16, body, 0)
    else:
        def row(i, _):
            def col(j, _):
                ref[i, pl.ds(j * 16, 16)] = val16
                return 0
            return lax.fori_loop(0, ncols // 16, col, 0)
        lax.fori_loop(0, nrows, row, 0)


def _make_agg(with_cnt: bool):
    mesh = plsc.VectorSubcoreMesh(core_axis_name="c", subcore_axis_name="s")
    out_type = [jax.ShapeDtypeStruct((NC, N_PAD, FEAT), jnp.float32)]
    scratch = [
        pltpu.VMEM((PHCH, CHUNK), jnp.int32),
        pltpu.VMEM((PHCH, CHUNK), jnp.int32),
        pltpu.VMEM((CHUNK, FEAT), jnp.float32),
        pltpu.VMEM((CHUNK, FEAT), jnp.float32),
        pltpu.VMEM_SHARED((N_PAD, FEAT), jnp.float32),
        pltpu.SemaphoreType.DMA,
        pltpu.SemaphoreType.DMA,
    ]
    if with_cnt:
        out_type.append(jax.ShapeDtypeStruct((NC, N_PAD), jnp.float32))
        scratch += [
            pltpu.VMEM((CHUNK,), jnp.float32),
            pltpu.VMEM((RPT,), jnp.float32),
            pltpu.VMEM_SHARED((N_PAD,), jnp.float32),
        ]

    def body(feat, src, dst, *rest):
        if with_cnt:
            (out_sum, out_cnt, didx, sidx, rows_a, rows_b, ssum, sem_a,
             sem_b, ones, zc, scnt) = rest
        else:
            out_sum, didx, sidx, rows_a, rows_b, ssum, sem_a, sem_b = rest
        cid = lax.axis_index("c")
        sid = lax.axis_index("s")
        wid = cid * NS + sid
        rb = sid * RPT

        zbuf = rows_a.at[pl.ds(0, ZCH)]
        _fill(zbuf, 0.0, ZCH, FEAT)
        for k in range(RPT // ZCH):
            pltpu.sync_copy(zbuf, ssum.at[pl.ds(rb + k * ZCH, ZCH), :])
        if with_cnt:
            _fill(ones, 1.0, CHUNK, None)
            _fill(zc, 0.0, RPT, None)
            pltpu.sync_copy(zc, scnt.at[pl.ds(rb, RPT)])

        plsc.subcore_barrier()

        def gather(i, rows, sem):
            return pltpu.async_copy(feat.at[sidx.at[i]], rows, sem)

        def scatter(i, rows):
            pltpu.sync_copy(rows, ssum.at[didx.at[i]], add=True)
            if with_cnt:
                pltpu.sync_copy(ones, scnt.at[didx.at[i]], add=True)

        for base in (0, PHCH):
            pltpu.sync_copy(dst.at[wid, pl.ds(base, PHCH)], didx)
            pltpu.sync_copy(src.at[wid, pl.ds(base, PHCH)], sidx)

            def group(g, _):
                go = g * UNROLL
                bufs = ((rows_a, sem_a), (rows_b, sem_b))
                descs = [None] * UNROLL
                descs[0] = gather(go, *bufs[0])
                for k in range(UNROLL):
                    if k + 1 < UNROLL:
                        descs[k + 1] = gather(go + k + 1, *bufs[(k + 1) % 2])
                    descs[k].wait()
                    scatter(go + k, bufs[k % 2][0])
                return 0

            lax.fori_loop(0, PHCH // UNROLL, group, 0)
        plsc.subcore_barrier()

        pltpu.sync_copy(ssum.at[pl.ds(rb, RPT), :],
                        out_sum.at[cid, pl.ds(rb, RPT), :])
        if with_cnt:
            pltpu.sync_copy(scnt.at[pl.ds(rb, RPT)],
                            out_cnt.at[cid, pl.ds(rb, RPT)])

    return pl.kernel(body, out_type=out_type, mesh=mesh,
                     scratch_types=scratch)


_agg_cnt = _make_agg(True)
_agg = _make_agg(False)

BR = 2048


def _dense_body(relu):
    def body(sp_ref, cp_ref, x_ref, wl_ref, wr_ref, b_ref, o_ref):
        c = jnp.clip(cp_ref[0] + cp_ref[1], 1.0, None)
        mean = (sp_ref[0] + sp_ref[1]) / c[:, None]
        acc = jnp.dot(mean, wl_ref[...], preferred_element_type=jnp.float32)
        acc = acc + jnp.dot(x_ref[...], wr_ref[...],
                            preferred_element_type=jnp.float32)
        acc = acc + b_ref[...]
        if relu:
            acc = jnp.maximum(acc, 0.0)
        o_ref[...] = acc
    return body


def _dense_layer(sp, cp, x, wl, wr, b, relu):
    return pl.pallas_call(
        _dense_body(relu),
        grid=(N_PAD // BR,),
        in_specs=[
            pl.BlockSpec((NC, BR, FEAT), lambda i: (0, i, 0)),
            pl.BlockSpec((NC, BR), lambda i: (0, i)),
            pl.BlockSpec((BR, FEAT), lambda i: (i, 0)),
            pl.BlockSpec((FEAT, FEAT), lambda i: (0, 0)),
            pl.BlockSpec((FEAT, FEAT), lambda i: (0, 0)),
            pl.BlockSpec((1, FEAT), lambda i: (0, 0)),
        ],
        out_specs=pl.BlockSpec((BR, FEAT), lambda i: (i, 0)),
        out_shape=jax.ShapeDtypeStruct((N_PAD, FEAT), jnp.float32),
    )(sp, cp, x, wl, wr, b)


def kernel(x, edge_index, Wl1, Wr1, b1, Wl2, Wr2, b2):
    ei = edge_index.astype(jnp.int32)
    rpw = N_EDGES // NW
    npad = EPW - rpw
    real = ei.reshape(2, NW, rpw)
    padv = jnp.broadcast_to(
        N_NODES + jnp.arange(npad, dtype=jnp.int32), (2, NW, npad))
    full = jnp.concatenate([real, padv], axis=2)
    src = full[0].reshape(NW, NCHUNKS, CHUNK)
    dst = full[1].reshape(NW, NCHUNKS, CHUNK)
    x_pad = jnp.pad(x, ((0, N_PAD - N_NODES), (0, 0)))
    sp1, cp = _agg_cnt(x_pad, src, dst)
    h = _dense_layer(sp1, cp, x_pad, Wl1, Wr1, b1.reshape(1, FEAT), True)
    sp2, = _agg(h, src, dst)
    out = _dense_layer(sp2, cp, h, Wl2, Wr2, b2.reshape(1, FEAT), False)
    return out[:N_NODES]
